# Initial kernel scaffold; baseline (speedup 1.0000x reference)
import jax
import jax.numpy as jnp
from jax.experimental import pallas as pl


def kernel(x, a):
    raise NotImplementedError("write your pallas kernel here")


# speedup vs baseline: 5.1154x; 5.1154x over previous
import functools

import jax
import jax.numpy as jnp
from jax import lax
from jax.experimental import pallas as pl
from jax.experimental.pallas import tpu as pltpu
from jax.experimental.pallas import tpu_sc as plsc

D = 128
N = 32768
NC = 2
NS = 16
L = 16
NW = NC * NS
COLS_PER_W = N // NW
CHUNK = 256
NCHUNK = COLS_PER_W // CHUNK
NGROUP = CHUNK // L
N_BISECT = 9
N_REFINE = 2
RU = 8


def _sc_body(x_hbm, a_hbm, out_hbm, xv, ov, av):
    wid = lax.axis_index("s") * NC + lax.axis_index("c")

    pltpu.sync_copy(a_hbm, av)
    a_vec = av[...]
    neg_aa = -1.0 / (1.0 + jnp.exp(-a_vec))

    def chunk_body(ci, _):
        base = wid * COLS_PER_W + ci * CHUNK
        pltpu.sync_copy(x_hbm.at[:, pl.ds(base, CHUNK)], xv)

        def group_body(g, _):
            c0 = g * L

            def max_body(i, m):
                r = i * RU
                for k in range(RU):
                    m = jnp.maximum(m, xv[r + k, pl.ds(c0, L)])
                return m

            m = lax.fori_loop(
                0, D // RU, max_body, jnp.full((L,), -3.0e38, jnp.float32)
            )

            lo = m - 1.0
            hi = m - (1.0 / D)

            def bisect_body(j, lohi):
                blo, bhi = lohi
                mid = 0.5 * (blo + bhi)

                def f_body(i, acc):
                    a0, a1 = acc
                    r = i * RU
                    for k in range(RU):
                        v = xv[r + k, pl.ds(c0, L)]
                        rl = jnp.maximum(v - mid, 0.0)
                        if k % 2 == 0:
                            a0 = a0 + rl
                        else:
                            a1 = a1 + rl
                    return (a0, a1)

                z = jnp.zeros((L,), jnp.float32)
                s0, s1 = lax.fori_loop(0, D // RU, f_body, (z, z))
                pred = (s0 + s1) > 1.0
                blo = jnp.where(pred, mid, blo)
                bhi = jnp.where(pred, bhi, mid)
                return (blo, bhi)

            lo, hi = lax.fori_loop(0, N_BISECT, bisect_body, (lo, hi))

            t = lo
            for _ in range(N_REFINE):

                def ref_body(i, acc, t=t):
                    s, c = acc
                    r = i * RU
                    for k in range(RU):
                        v = xv[r + k, pl.ds(c0, L)]
                        gt = v > t
                        s = s + jnp.where(gt, v, 0.0)
                        c = c + jnp.where(gt, 1.0, 0.0)
                    return (s, c)

                z = jnp.zeros((L,), jnp.float32)
                s, c = lax.fori_loop(0, D // RU, ref_body, (z, z))
                t = (s - 1.0) / c

            def out_body(i, _):
                r = i * RU
                for k in range(RU):
                    v = xv[r + k, pl.ds(c0, L)]
                    p = jnp.maximum(v - t, 0.0)
                    ov[r + k, pl.ds(c0, L)] = jnp.exp(neg_aa * p)
                return 0

            lax.fori_loop(0, D // RU, out_body, 0)
            return 0

        lax.fori_loop(0, NGROUP, group_body, 0)
        pltpu.sync_copy(ov, out_hbm.at[:, pl.ds(base, CHUNK)])
        return 0

    lax.fori_loop(0, NCHUNK, chunk_body, 0)


@functools.partial(jax.jit, static_argnames=())
def kernel(x, a):
    a_vec = jnp.broadcast_to(a.astype(jnp.float32), (L,))
    run = pl.kernel(
        _sc_body,
        mesh=plsc.VectorSubcoreMesh(core_axis_name="c", subcore_axis_name="s"),
        out_type=jax.ShapeDtypeStruct((D, N), jnp.float32),
        scratch_types=[
            pltpu.VMEM((D, CHUNK), jnp.float32),
            pltpu.VMEM((D, CHUNK), jnp.float32),
            pltpu.VMEM((L,), jnp.float32),
        ],
    )
    return run(x, a_vec)


# speedup vs baseline: 5.5951x; 1.0938x over previous
import functools

import jax
import jax.numpy as jnp
from jax import lax
from jax.experimental import pallas as pl
from jax.experimental.pallas import tpu as pltpu
from jax.experimental.pallas import tpu_sc as plsc

D = 128
N = 32768
NC = 2
NS = 16
L = 16
NW = NC * NS
COLS_PER_W = N // NW
CHUNK = 256
NCHUNK = COLS_PER_W // CHUNK
NGROUP = CHUNK // L
N_BISECT = 9
N_REFINE = 1
RU = 16


def _sc_body(x_hbm, a_hbm, out_hbm, xv, ov, av):
    wid = lax.axis_index("s") * NC + lax.axis_index("c")

    pltpu.sync_copy(a_hbm, av)
    a_vec = av[...]
    neg_aa = -1.0 / (1.0 + jnp.exp(-a_vec))

    def chunk_body(ci, _):
        base = wid * COLS_PER_W + ci * CHUNK
        pltpu.sync_copy(x_hbm.at[:, pl.ds(base, CHUNK)], xv)

        def group_body(g, _):
            c0 = g * L

            def max_body(i, m):
                r = i * RU
                for k in range(RU):
                    m = jnp.maximum(m, xv[r + k, pl.ds(c0, L)])
                return m

            m = lax.fori_loop(
                0, D // RU, max_body, jnp.full((L,), -3.0e38, jnp.float32)
            )

            lo = m - 1.0
            hi = m - (1.0 / D)

            def bisect_body(j, lohi):
                blo, bhi = lohi
                mid = 0.5 * (blo + bhi)

                def f_body(i, acc):
                    acc = list(acc)
                    r = i * RU
                    for k in range(RU):
                        v = xv[r + k, pl.ds(c0, L)]
                        rl = jnp.maximum(v - mid, 0.0)
                        acc[k % 4] = acc[k % 4] + rl
                    return tuple(acc)

                z = jnp.zeros((L,), jnp.float32)
                s0, s1, s2, s3 = lax.fori_loop(
                    0, D // RU, f_body, (z, z, z, z)
                )
                pred = ((s0 + s1) + (s2 + s3)) > 1.0
                blo = jnp.where(pred, mid, blo)
                bhi = jnp.where(pred, bhi, mid)
                return (blo, bhi)

            lo, hi = lax.fori_loop(0, N_BISECT, bisect_body, (lo, hi))

            t = lo
            for _ in range(N_REFINE):

                def ref_body(i, acc, t=t):
                    s0, s1, c0a, c1a = acc
                    r = i * RU
                    for k in range(RU):
                        v = xv[r + k, pl.ds(c0, L)]
                        gt = v > t
                        if k % 2 == 0:
                            s0 = s0 + jnp.where(gt, v, 0.0)
                            c0a = c0a + jnp.where(gt, 1.0, 0.0)
                        else:
                            s1 = s1 + jnp.where(gt, v, 0.0)
                            c1a = c1a + jnp.where(gt, 1.0, 0.0)
                    return (s0, s1, c0a, c1a)

                z = jnp.zeros((L,), jnp.float32)
                s0, s1, ca, cb = lax.fori_loop(
                    0, D // RU, ref_body, (z, z, z, z)
                )
                t = ((s0 + s1) - 1.0) / (ca + cb)

            def out_body(i, _):
                r = i * RU
                for k in range(RU):
                    v = xv[r + k, pl.ds(c0, L)]
                    p = jnp.maximum(v - t, 0.0)
                    ov[r + k, pl.ds(c0, L)] = jnp.exp(neg_aa * p)
                return 0

            lax.fori_loop(0, D // RU, out_body, 0)
            return 0

        lax.fori_loop(0, NGROUP, group_body, 0)
        pltpu.sync_copy(ov, out_hbm.at[:, pl.ds(base, CHUNK)])
        return 0

    lax.fori_loop(0, NCHUNK, chunk_body, 0)


@functools.partial(jax.jit, static_argnames=())
def kernel(x, a):
    a_vec = jnp.broadcast_to(a.astype(jnp.float32), (L,))
    run = pl.kernel(
        _sc_body,
        mesh=plsc.VectorSubcoreMesh(core_axis_name="c", subcore_axis_name="s"),
        out_type=jax.ShapeDtypeStruct((D, N), jnp.float32),
        scratch_types=[
            pltpu.VMEM((D, CHUNK), jnp.float32),
            pltpu.VMEM((D, CHUNK), jnp.float32),
            pltpu.VMEM((L,), jnp.float32),
        ],
    )
    return run(x, a_vec)


# speedup vs baseline: 5.6800x; 1.0152x over previous
import functools

import jax
import jax.numpy as jnp
from jax import lax
from jax.experimental import pallas as pl
from jax.experimental.pallas import tpu as pltpu
from jax.experimental.pallas import tpu_sc as plsc

D = 128
N = 32768
NC = 2
NS = 16
L = 16
NW = NC * NS
COLS_PER_W = N // NW
CHUNK = 256
NCHUNK = COLS_PER_W // CHUNK
NGROUP = CHUNK // L
N_BISECT = 10
N_REFINE = 0
RU = 16


def _sc_body(x_hbm, a_hbm, out_hbm, xv, ov, av):
    wid = lax.axis_index("s") * NC + lax.axis_index("c")

    pltpu.sync_copy(a_hbm, av)
    a_vec = av[...]
    neg_aa = -1.0 / (1.0 + jnp.exp(-a_vec))

    def chunk_body(ci, _):
        base = wid * COLS_PER_W + ci * CHUNK
        pltpu.sync_copy(x_hbm.at[:, pl.ds(base, CHUNK)], xv)

        def group_body(g, _):
            c0 = g * L

            def max_body(i, m):
                r = i * RU
                for k in range(RU):
                    m = jnp.maximum(m, xv[r + k, pl.ds(c0, L)])
                return m

            m = lax.fori_loop(
                0, D // RU, max_body, jnp.full((L,), -3.0e38, jnp.float32)
            )

            lo = m - 1.0
            hi = m - (1.0 / D)

            def bisect_body(j, lohi):
                blo, bhi = lohi
                mid = 0.5 * (blo + bhi)

                def f_body(i, acc):
                    acc = list(acc)
                    r = i * RU
                    for k in range(RU):
                        v = xv[r + k, pl.ds(c0, L)]
                        rl = jnp.maximum(v - mid, 0.0)
                        acc[k % 4] = acc[k % 4] + rl
                    return tuple(acc)

                z = jnp.zeros((L,), jnp.float32)
                s0, s1, s2, s3 = lax.fori_loop(
                    0, D // RU, f_body, (z, z, z, z)
                )
                pred = ((s0 + s1) + (s2 + s3)) > 1.0
                blo = jnp.where(pred, mid, blo)
                bhi = jnp.where(pred, bhi, mid)
                return (blo, bhi)

            lo, hi = lax.fori_loop(0, N_BISECT, bisect_body, (lo, hi))

            t = 0.5 * (lo + hi)

            def out_body(i, _):
                r = i * RU
                for k in range(RU):
                    v = xv[r + k, pl.ds(c0, L)]
                    p = jnp.maximum(v - t, 0.0)
                    ov[r + k, pl.ds(c0, L)] = jnp.exp(neg_aa * p)
                return 0

            lax.fori_loop(0, D // RU, out_body, 0)
            return 0

        lax.fori_loop(0, NGROUP, group_body, 0)
        pltpu.sync_copy(ov, out_hbm.at[:, pl.ds(base, CHUNK)])
        return 0

    lax.fori_loop(0, NCHUNK, chunk_body, 0)


@functools.partial(jax.jit, static_argnames=())
def kernel(x, a):
    a_vec = jnp.broadcast_to(a.astype(jnp.float32), (L,))
    run = pl.kernel(
        _sc_body,
        mesh=plsc.VectorSubcoreMesh(core_axis_name="c", subcore_axis_name="s"),
        out_type=jax.ShapeDtypeStruct((D, N), jnp.float32),
        scratch_types=[
            pltpu.VMEM((D, CHUNK), jnp.float32),
            pltpu.VMEM((D, CHUNK), jnp.float32),
            pltpu.VMEM((L,), jnp.float32),
        ],
    )
    return run(x, a_vec)


# speedup vs baseline: 10.8414x; 1.9087x over previous
import functools

import jax
import jax.numpy as jnp
from jax import lax
from jax.experimental import pallas as pl
from jax.experimental.pallas import tpu as pltpu
from jax.experimental.pallas import tpu_sc as plsc

D = 128
N = 32768
NC = 2
NS = 16
L = 16
NW = NC * NS
COLS_PER_W = N // NW
CHUNK = 256
NCHUNK = COLS_PER_W // CHUNK
NGROUP = CHUNK // L
N_BISECT = 10
N_REFINE = 0
RU = 16


def _sc_body(x_hbm, a_hbm, out_hbm, xv, ov, av):
    wid = lax.axis_index("s") * NC + lax.axis_index("c")

    pltpu.sync_copy(a_hbm, av)
    a_vec = av[...]
    neg_aa = -1.0 / (1.0 + jnp.exp(-a_vec))

    def chunk_body(ci, _):
        base = wid * COLS_PER_W + ci * CHUNK
        pltpu.sync_copy(x_hbm.at[:, pl.ds(base, CHUNK)], xv)

        def group_body(g, _):
            c0 = g * L

            def max_body(i, m):
                r = i * RU
                for k in range(RU):
                    m = jnp.maximum(m, xv[r + k, pl.ds(c0, L)])
                return m

            m = lax.fori_loop(
                0, D // RU, max_body, jnp.full((L,), -3.0e38, jnp.float32)
            )

            lo = m - 1.0
            hi = m - (1.0 / D)

            def bisect_body(j, lohi):
                blo, bhi = lohi
                mid = 0.5 * (blo + bhi)

                def f_body(i, acc):
                    acc = list(acc)
                    r = i * RU
                    for k in range(RU):
                        v = xv[r + k, pl.ds(c0, L)]
                        rl = jnp.maximum(v - mid, 0.0)
                        acc[k % 4] = acc[k % 4] + rl
                    return tuple(acc)

                z = jnp.zeros((L,), jnp.float32)
                s0, s1, s2, s3 = lax.fori_loop(
                    0, D // RU, f_body, (z, z, z, z)
                )
                pred = ((s0 + s1) + (s2 + s3)) > 1.0
                blo = jnp.where(pred, mid, blo)
                bhi = jnp.where(pred, bhi, mid)
                return (blo, bhi)

            lo, hi = lax.fori_loop(0, N_BISECT, bisect_body, (lo, hi))

            t = 0.5 * (lo + hi)

            @plsc.parallel_loop(0, D, step=1, unroll=8)
            def _(r):
                v = xv[r, pl.ds(c0, L)]
                p = jnp.maximum(v - t, 0.0)
                ov[r, pl.ds(c0, L)] = jnp.exp(neg_aa * p)

            return 0

        lax.fori_loop(0, NGROUP, group_body, 0)
        pltpu.sync_copy(ov, out_hbm.at[:, pl.ds(base, CHUNK)])
        return 0

    lax.fori_loop(0, NCHUNK, chunk_body, 0)


@functools.partial(jax.jit, static_argnames=())
def kernel(x, a):
    a_vec = jnp.broadcast_to(a.astype(jnp.float32), (L,))
    run = pl.kernel(
        _sc_body,
        mesh=plsc.VectorSubcoreMesh(core_axis_name="c", subcore_axis_name="s"),
        out_type=jax.ShapeDtypeStruct((D, N), jnp.float32),
        scratch_types=[
            pltpu.VMEM((D, CHUNK), jnp.float32),
            pltpu.VMEM((D, CHUNK), jnp.float32),
            pltpu.VMEM((L,), jnp.float32),
        ],
    )
    return run(x, a_vec)


# speedup vs baseline: 11.1610x; 1.0295x over previous
import functools

import jax
import jax.numpy as jnp
from jax import lax
from jax.experimental import pallas as pl
from jax.experimental.pallas import tpu as pltpu
from jax.experimental.pallas import tpu_sc as plsc

D = 128
N = 32768
NC = 2
NS = 16
L = 16
NW = NC * NS
COLS_PER_W = N // NW
CHUNK = 256
NCHUNK = COLS_PER_W // CHUNK
NGROUP = CHUNK // L
N_BISECT = 10
N_REFINE = 0
RU = 16


def _sc_body(x_hbm, a_hbm, out_hbm, xv, ov, av):
    wid = lax.axis_index("s") * NC + lax.axis_index("c")

    pltpu.sync_copy(a_hbm, av)
    a_vec = av[...]
    neg_aa = -1.0 / (1.0 + jnp.exp(-a_vec))

    def chunk_body(ci, _):
        base = wid * COLS_PER_W + ci * CHUNK
        pltpu.sync_copy(x_hbm.at[:, pl.ds(base, CHUNK)], xv)

        def group_body(g, _):
            c0 = g * L

            neg_big = jnp.full((L,), -3.0e38, jnp.float32)

            @plsc.parallel_loop(
                0, D, step=RU, carry=(neg_big, neg_big, neg_big, neg_big)
            )
            def _max_loop(r, ms):
                ms = list(ms)
                for k in range(RU):
                    ms[k % 4] = jnp.maximum(ms[k % 4], xv[r + k, pl.ds(c0, L)])
                return tuple(ms)

            m0, m1, m2, m3 = _max_loop
            m = jnp.maximum(jnp.maximum(m0, m1), jnp.maximum(m2, m3))

            lo = m - 1.0
            hi = m - (1.0 / D)

            def bisect_body(j, lohi):
                blo, bhi = lohi
                mid = 0.5 * (blo + bhi)

                z = jnp.zeros((L,), jnp.float32)

                @plsc.parallel_loop(0, D, step=RU, carry=(z, z, z, z))
                def _f_loop(r, acc):
                    acc = list(acc)
                    for k in range(RU):
                        v = xv[r + k, pl.ds(c0, L)]
                        rl = jnp.maximum(v - mid, 0.0)
                        acc[k % 4] = acc[k % 4] + rl
                    return tuple(acc)

                s0, s1, s2, s3 = _f_loop
                pred = ((s0 + s1) + (s2 + s3)) > 1.0
                blo = jnp.where(pred, mid, blo)
                bhi = jnp.where(pred, bhi, mid)
                return (blo, bhi)

            lo, hi = lax.fori_loop(0, N_BISECT, bisect_body, (lo, hi))

            t = 0.5 * (lo + hi)

            @plsc.parallel_loop(0, D, step=1, unroll=8)
            def _(r):
                v = xv[r, pl.ds(c0, L)]
                p = jnp.maximum(v - t, 0.0)
                ov[r, pl.ds(c0, L)] = jnp.exp(neg_aa * p)

            return 0

        lax.fori_loop(0, NGROUP, group_body, 0)
        pltpu.sync_copy(ov, out_hbm.at[:, pl.ds(base, CHUNK)])
        return 0

    lax.fori_loop(0, NCHUNK, chunk_body, 0)


@functools.partial(jax.jit, static_argnames=())
def kernel(x, a):
    a_vec = jnp.broadcast_to(a.astype(jnp.float32), (L,))
    run = pl.kernel(
        _sc_body,
        mesh=plsc.VectorSubcoreMesh(core_axis_name="c", subcore_axis_name="s"),
        out_type=jax.ShapeDtypeStruct((D, N), jnp.float32),
        scratch_types=[
            pltpu.VMEM((D, CHUNK), jnp.float32),
            pltpu.VMEM((D, CHUNK), jnp.float32),
            pltpu.VMEM((L,), jnp.float32),
        ],
    )
    return run(x, a_vec)


# speedup vs baseline: 12.2954x; 1.1016x over previous
import functools

import jax
import jax.numpy as jnp
from jax import lax
from jax.experimental import pallas as pl
from jax.experimental.pallas import tpu as pltpu
from jax.experimental.pallas import tpu_sc as plsc

D = 128
N = 32768
NC = 2
NS = 16
L = 16
NW = NC * NS
COLS_PER_W = N // NW
CHUNK = 128
NCHUNK = COLS_PER_W // CHUNK
NBUF = 3
NGROUP = CHUNK // L
N_BISECT = 10
RU = 16


def _sc_body(x_hbm, a_hbm, out_hbm, xv, av, in_sems, out_sems):
    wid = lax.axis_index("s") * NC + lax.axis_index("c")
    col0 = wid * COLS_PER_W

    pltpu.sync_copy(a_hbm, av)
    a_vec = av[...]
    neg_aa = -1.0 / (1.0 + jnp.exp(-a_vec))

    def in_slice(ci):
        return x_hbm.at[:, pl.ds(col0 + ci * CHUNK, CHUNK)]

    def out_slice(ci):
        return out_hbm.at[:, pl.ds(col0 + ci * CHUNK, CHUNK)]

    pltpu.async_copy(in_slice(0), xv.at[0], in_sems.at[0])
    pltpu.async_copy(in_slice(1), xv.at[1], in_sems.at[1])

    def chunk_body(ci, _):
        b = lax.rem(ci, NBUF)
        pltpu.make_async_copy(in_slice(ci), xv.at[b], in_sems.at[b]).wait()

        def group_body(g, _):
            c0 = g * L
            neg_big = jnp.full((L,), -3.0e38, jnp.float32)

            @plsc.parallel_loop(
                0, D, step=RU, carry=(neg_big, neg_big, neg_big, neg_big)
            )
            def _max_loop(r, ms):
                ms = list(ms)
                for k in range(RU):
                    ms[k % 4] = jnp.maximum(
                        ms[k % 4], xv[b, r + k, pl.ds(c0, L)]
                    )
                return tuple(ms)

            m0, m1, m2, m3 = _max_loop
            m = jnp.maximum(jnp.maximum(m0, m1), jnp.maximum(m2, m3))

            lo = m - 1.0
            hi = m - (1.0 / D)

            def bisect_body(j, lohi):
                blo, bhi = lohi
                mid = 0.5 * (blo + bhi)
                z = jnp.zeros((L,), jnp.float32)

                @plsc.parallel_loop(0, D, step=RU, carry=(z, z, z, z))
                def _f_loop(r, acc):
                    acc = list(acc)
                    for k in range(RU):
                        v = xv[b, r + k, pl.ds(c0, L)]
                        rl = jnp.maximum(v - mid, 0.0)
                        acc[k % 4] = acc[k % 4] + rl
                    return tuple(acc)

                s0, s1, s2, s3 = _f_loop
                pred = ((s0 + s1) + (s2 + s3)) > 1.0
                blo = jnp.where(pred, mid, blo)
                bhi = jnp.where(pred, bhi, mid)
                return (blo, bhi)

            lo, hi = lax.fori_loop(0, N_BISECT, bisect_body, (lo, hi))
            t = 0.5 * (lo + hi)

            @plsc.parallel_loop(0, D, step=1, unroll=8)
            def _(r):
                v = xv[b, r, pl.ds(c0, L)]
                p = jnp.maximum(v - t, 0.0)
                xv[b, r, pl.ds(c0, L)] = jnp.exp(neg_aa * p)

            return 0

        lax.fori_loop(0, NGROUP, group_body, 0)
        pltpu.async_copy(xv.at[b], out_slice(ci), out_sems.at[b])

        @pl.when(ci + 2 < NCHUNK)
        def _():
            b2 = lax.rem(ci + 2, NBUF)

            @pl.when(ci >= 1)
            def _():
                pltpu.make_async_copy(
                    xv.at[b2], out_slice(ci - 1), out_sems.at[b2]
                ).wait()

            pltpu.async_copy(in_slice(ci + 2), xv.at[b2], in_sems.at[b2])

        return 0

    lax.fori_loop(0, NCHUNK, chunk_body, 0)

    for ci in range(NCHUNK - NBUF, NCHUNK):
        b = ci % NBUF
        pltpu.make_async_copy(xv.at[b], out_slice(ci), out_sems.at[b]).wait()


@functools.partial(jax.jit, static_argnames=())
def kernel(x, a):
    a_vec = jnp.broadcast_to(a.astype(jnp.float32), (L,))
    run = pl.kernel(
        _sc_body,
        mesh=plsc.VectorSubcoreMesh(core_axis_name="c", subcore_axis_name="s"),
        out_type=jax.ShapeDtypeStruct((D, N), jnp.float32),
        scratch_types=[
            pltpu.VMEM((NBUF, D, CHUNK), jnp.float32),
            pltpu.VMEM((L,), jnp.float32),
            pltpu.SemaphoreType.DMA((NBUF,)),
            pltpu.SemaphoreType.DMA((NBUF,)),
        ],
    )
    return run(x, a_vec)


# speedup vs baseline: 13.8905x; 1.1297x over previous
import functools

import jax
import jax.numpy as jnp
from jax import lax
from jax.experimental import pallas as pl
from jax.experimental.pallas import tpu as pltpu
from jax.experimental.pallas import tpu_sc as plsc

D = 128
N = 32768
NC = 2
NS = 16
L = 16
NW = NC * NS
COLS_PER_W = N // NW
CHUNK = 128
NCHUNK = COLS_PER_W // CHUNK
NBUF = 3
NGROUP = CHUNK // L
N_BISECT = 8
RU = 16


def _sc_body(x_hbm, a_hbm, out_hbm, xv, av, in_sems, out_sems):
    wid = lax.axis_index("s") * NC + lax.axis_index("c")
    col0 = wid * COLS_PER_W

    pltpu.sync_copy(a_hbm, av)
    a_vec = av[...]
    neg_aa = -1.0 / (1.0 + jnp.exp(-a_vec))

    def in_slice(ci):
        return x_hbm.at[:, pl.ds(col0 + ci * CHUNK, CHUNK)]

    def out_slice(ci):
        return out_hbm.at[:, pl.ds(col0 + ci * CHUNK, CHUNK)]

    pltpu.async_copy(in_slice(0), xv.at[0], in_sems.at[0])
    pltpu.async_copy(in_slice(1), xv.at[1], in_sems.at[1])

    def chunk_body(ci, _):
        b = lax.rem(ci, NBUF)
        pltpu.make_async_copy(in_slice(ci), xv.at[b], in_sems.at[b]).wait()

        def group_body(g, _):
            c0 = g * L
            neg_big = jnp.full((L,), -3.0e38, jnp.float32)

            @plsc.parallel_loop(
                0, D, step=RU, carry=(neg_big, neg_big, neg_big, neg_big)
            )
            def _max_loop(r, ms):
                ms = list(ms)
                for k in range(RU):
                    ms[k % 4] = jnp.maximum(
                        ms[k % 4], xv[b, r + k, pl.ds(c0, L)]
                    )
                return tuple(ms)

            m0, m1, m2, m3 = _max_loop
            m = jnp.maximum(jnp.maximum(m0, m1), jnp.maximum(m2, m3))

            lo = m - 1.0
            hi = m - (1.0 / D)

            def bisect_body(j, lohi):
                blo, bhi = lohi
                mid = 0.5 * (blo + bhi)
                z = jnp.zeros((L,), jnp.float32)

                @plsc.parallel_loop(0, D, step=RU, carry=(z, z, z, z))
                def _f_loop(r, acc):
                    acc = list(acc)
                    for k in range(RU):
                        v = xv[b, r + k, pl.ds(c0, L)]
                        rl = jnp.maximum(v - mid, 0.0)
                        acc[k % 4] = acc[k % 4] + rl
                    return tuple(acc)

                s0, s1, s2, s3 = _f_loop
                pred = ((s0 + s1) + (s2 + s3)) > 1.0
                blo = jnp.where(pred, mid, blo)
                bhi = jnp.where(pred, bhi, mid)
                return (blo, bhi)

            lo, hi = lax.fori_loop(0, N_BISECT, bisect_body, (lo, hi))
            t = 0.5 * (lo + hi)

            @plsc.parallel_loop(0, D, step=1, unroll=8)
            def _(r):
                v = xv[b, r, pl.ds(c0, L)]
                p = jnp.maximum(v - t, 0.0)
                xv[b, r, pl.ds(c0, L)] = jnp.exp(neg_aa * p)

            return 0

        lax.fori_loop(0, NGROUP, group_body, 0)
        pltpu.async_copy(xv.at[b], out_slice(ci), out_sems.at[b])

        @pl.when(ci + 2 < NCHUNK)
        def _():
            b2 = lax.rem(ci + 2, NBUF)

            @pl.when(ci >= 1)
            def _():
                pltpu.make_async_copy(
                    xv.at[b2], out_slice(ci - 1), out_sems.at[b2]
                ).wait()

            pltpu.async_copy(in_slice(ci + 2), xv.at[b2], in_sems.at[b2])

        return 0

    lax.fori_loop(0, NCHUNK, chunk_body, 0)

    for ci in range(NCHUNK - NBUF, NCHUNK):
        b = ci % NBUF
        pltpu.make_async_copy(xv.at[b], out_slice(ci), out_sems.at[b]).wait()


@functools.partial(jax.jit, static_argnames=())
def kernel(x, a):
    a_vec = jnp.broadcast_to(a.astype(jnp.float32), (L,))
    run = pl.kernel(
        _sc_body,
        mesh=plsc.VectorSubcoreMesh(core_axis_name="c", subcore_axis_name="s"),
        out_type=jax.ShapeDtypeStruct((D, N), jnp.float32),
        scratch_types=[
            pltpu.VMEM((NBUF, D, CHUNK), jnp.float32),
            pltpu.VMEM((L,), jnp.float32),
            pltpu.SemaphoreType.DMA((NBUF,)),
            pltpu.SemaphoreType.DMA((NBUF,)),
        ],
    )
    return run(x, a_vec)


# speedup vs baseline: 36.6043x; 2.6352x over previous
import functools

import jax
import jax.numpy as jnp
from jax import lax
from jax.experimental import pallas as pl
from jax.experimental.pallas import tpu as pltpu
from jax.experimental.pallas import tpu_sc as plsc

D = 128
N = 32768
NC = 2
NS = 16
L = 16
NW = NC * NS
COLS_PER_W = N // NW
CHUNK = 128
NCHUNK = COLS_PER_W // CHUNK
NBUF = 3
NGROUP = CHUNK // L
N_BISECT = 8
RU = 16


def _sc_body(x_hbm, a_hbm, out_hbm, xv, av, in_sems, out_sems):
    wid = lax.axis_index("s") * NC + lax.axis_index("c")
    col0 = wid * COLS_PER_W

    pltpu.sync_copy(a_hbm, av)
    a_vec = av[...]
    neg_aa = -1.0 / (1.0 + jnp.exp(-a_vec))

    def in_slice(ci):
        return x_hbm.at[:, pl.ds(col0 + ci * CHUNK, CHUNK)]

    def out_slice(ci):
        return out_hbm.at[:, pl.ds(col0 + ci * CHUNK, CHUNK)]

    pltpu.async_copy(in_slice(0), xv.at[0], in_sems.at[0])
    pltpu.async_copy(in_slice(1), xv.at[1], in_sems.at[1])

    def chunk_body(ci, _):
        b = lax.rem(ci, NBUF)
        pltpu.make_async_copy(in_slice(ci), xv.at[b], in_sems.at[b]).wait()

        def group_body(g, _):
            c0 = g * L
            neg_big = jnp.full((L,), -3.0e38, jnp.float32)

            @plsc.parallel_loop(
                0, D, step=RU, carry=(neg_big, neg_big, neg_big, neg_big)
            )
            def _max_loop(r, ms):
                ms = list(ms)
                for k in range(RU):
                    ms[k % 4] = jnp.maximum(
                        ms[k % 4], xv[b, r + k, pl.ds(c0, L)]
                    )
                return tuple(ms)

            m0, m1, m2, m3 = _max_loop
            m = jnp.maximum(jnp.maximum(m0, m1), jnp.maximum(m2, m3))

            lo = m - 1.0
            hi = m - (1.0 / D)

            def bisect_body(j, lohi):
                blo, bhi = lohi
                mid = 0.5 * (blo + bhi)
                z = jnp.zeros((L,), jnp.float32)

                @plsc.parallel_loop(0, D, step=RU, carry=(z, z, z, z))
                def _f_loop(r, acc):
                    acc = list(acc)
                    for k in range(RU):
                        v = xv[b, r + k, pl.ds(c0, L)]
                        rl = jnp.maximum(v - mid, 0.0)
                        acc[k % 4] = acc[k % 4] + rl
                    return tuple(acc)

                s0, s1, s2, s3 = _f_loop
                pred = ((s0 + s1) + (s2 + s3)) > 1.0
                blo = jnp.where(pred, mid, blo)
                bhi = jnp.where(pred, bhi, mid)
                return (blo, bhi)

            lo, hi = lax.fori_loop(0, N_BISECT, bisect_body, (lo, hi))
            t = 0.5 * (lo + hi)

            @plsc.parallel_loop(0, D, step=1, unroll=8)
            def _(r):
                v = xv[b, r, pl.ds(c0, L)]
                p = jnp.maximum(v - t, 0.0)
                xv[b, r, pl.ds(c0, L)] = jnp.exp(neg_aa * p)

            return 0

        lax.fori_loop(0, NGROUP, group_body, 0)
        pltpu.async_copy(xv.at[b], out_slice(ci), out_sems.at[b])

        @pl.when(ci + 2 < NCHUNK)
        def _():
            b2 = lax.rem(ci + 2, NBUF)

            @pl.when(ci >= 1)
            def _():
                pltpu.make_async_copy(
                    xv.at[b2], out_slice(ci - 1), out_sems.at[b2]
                ).wait()

            pltpu.async_copy(in_slice(ci + 2), xv.at[b2], in_sems.at[b2])

        return 0

    lax.fori_loop(0, NCHUNK, chunk_body, 0)

    for ci in range(NCHUNK - NBUF, NCHUNK):
        b = ci % NBUF
        pltpu.make_async_copy(xv.at[b], out_slice(ci), out_sems.at[b]).wait()


TC_BN = 2048


def _tc_body(a_sref, x_ref, o_ref):
    a_val = a_sref[0]
    neg_aa = -1.0 / (1.0 + jnp.exp(-a_val))
    x = x_ref[...]
    m = jnp.max(x, axis=0, keepdims=True)
    lo = m - 1.0
    hi = m - (1.0 / D)

    def bisect_body(j, lohi):
        blo, bhi = lohi
        mid = 0.5 * (blo + bhi)
        f = jnp.sum(jnp.maximum(x - mid, 0.0), axis=0, keepdims=True)
        pred = f > 1.0
        return (jnp.where(pred, mid, blo), jnp.where(pred, bhi, mid))

    lo, hi = lax.fori_loop(0, N_BISECT, bisect_body, (lo, hi))
    t = 0.5 * (lo + hi)
    o_ref[...] = jnp.exp(neg_aa * jnp.maximum(x - t, 0.0))


def _tc_half(x_cols, a):
    ncols = x_cols.shape[1]
    a_arr = jnp.reshape(a.astype(jnp.float32), (1,))
    return pl.pallas_call(
        _tc_body,
        grid=(ncols // TC_BN,),
        in_specs=[
            pl.BlockSpec(memory_space=pltpu.SMEM),
            pl.BlockSpec((D, TC_BN), lambda i: (0, i)),
        ],
        out_specs=pl.BlockSpec((D, TC_BN), lambda i: (0, i)),
        out_shape=jax.ShapeDtypeStruct((D, ncols), jnp.float32),
    )(a_arr, x_cols)


@functools.partial(jax.jit, static_argnames=())
def kernel(x, a):
    return _tc_half(x, a)
    a_vec = jnp.broadcast_to(a.astype(jnp.float32), (L,))
    run = pl.kernel(
        _sc_body,
        mesh=plsc.VectorSubcoreMesh(core_axis_name="c", subcore_axis_name="s"),
        out_type=jax.ShapeDtypeStruct((D, N), jnp.float32),
        scratch_types=[
            pltpu.VMEM((NBUF, D, CHUNK), jnp.float32),
            pltpu.VMEM((L,), jnp.float32),
            pltpu.SemaphoreType.DMA((NBUF,)),
            pltpu.SemaphoreType.DMA((NBUF,)),
        ],
    )
    return run(x, a_vec)
